# baseline (speedup 1.0000x reference)
import functools

import jax
import jax.numpy as jnp
from jax import lax
from jax.experimental import pallas as pl
from jax.experimental.pallas import tpu as pltpu
from jax.experimental.pallas import tpu_sc as plsc

_S = 8192
_D = 1024
_WORDS = _S * _D
_NC = 2
_NS = 16
_NW = _NC * _NS
_W_WORDS = _WORDS // _NW
_CHUNK = 16384
_N_CHUNKS = _W_WORDS // _CHUNK


def _sc_body(x_hbm, e_hbm, o_hbm, xb, eb):
    wid = lax.axis_index("s") * _NC + lax.axis_index("c")
    base = wid * _W_WORDS

    def chunk_body(g, carry):
        off = base + g * _CHUNK
        pltpu.sync_copy(x_hbm.at[pl.ds(off, _CHUNK)], xb)
        pltpu.sync_copy(e_hbm.at[pl.ds(off, _CHUNK)], eb)

        def vec_body(i, c):
            s = pl.ds(i * 16, 16)
            xb[s] = xb[s] + eb[s]
            return c

        lax.fori_loop(0, _CHUNK // 16, vec_body, 0)
        pltpu.sync_copy(xb, o_hbm.at[pl.ds(off, _CHUNK)])
        return carry

    lax.fori_loop(0, _N_CHUNKS, chunk_body, 0)


_sc_add = functools.partial(
    pl.kernel,
    mesh=plsc.VectorSubcoreMesh(core_axis_name="c", subcore_axis_name="s"),
    out_type=jax.ShapeDtypeStruct((_WORDS,), jnp.float32),
    scratch_types=[
        pltpu.VMEM((_CHUNK,), jnp.float32),
        pltpu.VMEM((_CHUNK,), jnp.float32),
    ],
)(_sc_body)


def kernel(x, encoding):
    N, S, D = x.shape
    out = _sc_add(x.reshape(_WORDS), encoding.reshape(_WORDS))
    return out.reshape(N, S, D)


# speedup vs baseline: 1.6212x; 1.6212x over previous
import functools

import jax
import jax.numpy as jnp
from jax import lax
from jax.experimental import pallas as pl
from jax.experimental.pallas import tpu as pltpu
from jax.experimental.pallas import tpu_sc as plsc

_S = 8192
_D = 1024
_WORDS = _S * _D
_NC = 2
_NS = 16
_NW = _NC * _NS
_W_WORDS = _WORDS // _NW
_CHUNK = 16384
_N_CHUNKS = _W_WORDS // _CHUNK
_UNROLL = 8


def _sc_body(x_hbm, e_hbm, o_hbm, xb0, eb0, xb1, eb1, sx0, se0, so0, sx1, se1, so1):
    wid = lax.axis_index("s") * _NC + lax.axis_index("c")
    base = wid * _W_WORDS
    xbs = (xb0, xb1)
    ebs = (eb0, eb1)
    sxs = (sx0, sx1)
    ses = (se0, se1)
    sos = (so0, so1)

    def start_in(g, b):
        off = base + g * _CHUNK
        pltpu.make_async_copy(x_hbm.at[pl.ds(off, _CHUNK)], xbs[b], sxs[b]).start()
        pltpu.make_async_copy(e_hbm.at[pl.ds(off, _CHUNK)], ebs[b], ses[b]).start()

    def wait_in(b):
        pltpu.make_async_copy(x_hbm.at[pl.ds(0, _CHUNK)], xbs[b], sxs[b]).wait()
        pltpu.make_async_copy(e_hbm.at[pl.ds(0, _CHUNK)], ebs[b], ses[b]).wait()

    def start_out(g, b):
        off = base + g * _CHUNK
        pltpu.make_async_copy(xbs[b], o_hbm.at[pl.ds(off, _CHUNK)], sos[b]).start()

    def wait_out(b):
        pltpu.make_async_copy(xbs[b], o_hbm.at[pl.ds(0, _CHUNK)], sos[b]).wait()

    def compute(b):
        xb, eb = xbs[b], ebs[b]

        def vec_body(i, c):
            ibase = i * (16 * _UNROLL)
            for u in range(_UNROLL):
                s = pl.ds(ibase + u * 16, 16)
                xb[s] = xb[s] + eb[s]
            return c

        lax.fori_loop(0, _CHUNK // (16 * _UNROLL), vec_body, 0)

    start_in(0, 0)
    for g in range(_N_CHUNKS):
        b = g % 2
        if g + 1 < _N_CHUNKS:
            if g >= 1:
                wait_out(1 - b)
            start_in(g + 1, 1 - b)
        wait_in(b)
        compute(b)
        start_out(g, b)
    wait_out(0)
    wait_out(1)


_sc_add = functools.partial(
    pl.kernel,
    mesh=plsc.VectorSubcoreMesh(core_axis_name="c", subcore_axis_name="s"),
    out_type=jax.ShapeDtypeStruct((_WORDS,), jnp.float32),
    scratch_types=[
        pltpu.VMEM((_CHUNK,), jnp.float32),
        pltpu.VMEM((_CHUNK,), jnp.float32),
        pltpu.VMEM((_CHUNK,), jnp.float32),
        pltpu.VMEM((_CHUNK,), jnp.float32),
        pltpu.SemaphoreType.DMA,
        pltpu.SemaphoreType.DMA,
        pltpu.SemaphoreType.DMA,
        pltpu.SemaphoreType.DMA,
        pltpu.SemaphoreType.DMA,
        pltpu.SemaphoreType.DMA,
    ],
)(_sc_body)


def kernel(x, encoding):
    N, S, D = x.shape
    out = _sc_add(x.reshape(_WORDS), encoding.reshape(_WORDS))
    return out.reshape(N, S, D)


# speedup vs baseline: 3.4850x; 2.1496x over previous
import functools

import jax
import jax.numpy as jnp
from jax import lax
from jax.experimental import pallas as pl
from jax.experimental.pallas import tpu as pltpu
from jax.experimental.pallas import tpu_sc as plsc

_S = 8192
_D = 1024
_NC = 2
_NS = 16
_NW = _NC * _NS
_W_ROWS = _S // _NW
_CROWS = 16
_N_CHUNKS = _W_ROWS // _CROWS
_VPR = _D // 16


def _sc_body(x_hbm, e_hbm, o_hbm, xb0, eb0, xb1, eb1, sx0, se0, so0, sx1, se1, so1):
    wid = lax.axis_index("s") * _NC + lax.axis_index("c")
    base = wid * _W_ROWS
    xbs = (xb0, xb1)
    ebs = (eb0, eb1)
    sxs = (sx0, sx1)
    ses = (se0, se1)
    sos = (so0, so1)

    def start_in(g, b):
        r0 = base + g * _CROWS
        pltpu.make_async_copy(x_hbm.at[0, pl.ds(r0, _CROWS)], xbs[b], sxs[b]).start()
        pltpu.make_async_copy(e_hbm.at[pl.ds(r0, _CROWS)], ebs[b], ses[b]).start()

    def wait_in(b):
        pltpu.make_async_copy(x_hbm.at[0, pl.ds(0, _CROWS)], xbs[b], sxs[b]).wait()
        pltpu.make_async_copy(e_hbm.at[pl.ds(0, _CROWS)], ebs[b], ses[b]).wait()

    def start_out(g, b):
        r0 = base + g * _CROWS
        pltpu.make_async_copy(xbs[b], o_hbm.at[0, pl.ds(r0, _CROWS)], sos[b]).start()

    def wait_out(b):
        pltpu.make_async_copy(xbs[b], o_hbm.at[0, pl.ds(0, _CROWS)], sos[b]).wait()

    def compute(b):
        xb, eb = xbs[b], ebs[b]

        def vec_body(i, c):
            s = pl.ds(i * 16, 16)
            for r in range(_CROWS):
                xb[r, s] = xb[r, s] + eb[r, s]
            return c

        lax.fori_loop(0, _VPR, vec_body, 0)

    start_in(0, 0)
    for g in range(_N_CHUNKS):
        b = g % 2
        if g + 1 < _N_CHUNKS:
            if g >= 1:
                wait_out(1 - b)
            start_in(g + 1, 1 - b)
        wait_in(b)
        compute(b)
        start_out(g, b)
    wait_out(0)
    wait_out(1)


_sc_add = functools.partial(
    pl.kernel,
    mesh=plsc.VectorSubcoreMesh(core_axis_name="c", subcore_axis_name="s"),
    out_type=jax.ShapeDtypeStruct((1, _S, _D), jnp.float32),
    scratch_types=[
        pltpu.VMEM((_CROWS, _D), jnp.float32),
        pltpu.VMEM((_CROWS, _D), jnp.float32),
        pltpu.VMEM((_CROWS, _D), jnp.float32),
        pltpu.VMEM((_CROWS, _D), jnp.float32),
        pltpu.SemaphoreType.DMA,
        pltpu.SemaphoreType.DMA,
        pltpu.SemaphoreType.DMA,
        pltpu.SemaphoreType.DMA,
        pltpu.SemaphoreType.DMA,
        pltpu.SemaphoreType.DMA,
    ],
)(_sc_body)


def kernel(x, encoding):
    return _sc_add(x, encoding)


# speedup vs baseline: 4.2460x; 1.2183x over previous
import functools

import jax
import jax.numpy as jnp
from jax import lax
from jax.experimental import pallas as pl
from jax.experimental.pallas import tpu as pltpu
from jax.experimental.pallas import tpu_sc as plsc

_S = 8192
_D = 1024
_NC = 2
_NS = 16
_NW = _NC * _NS

_TC_ROWS = 6144
_SC_ROWS = _S - _TC_ROWS
_W_ROWS = _SC_ROWS // _NW
_CROWS = 16
_N_CHUNKS = _W_ROWS // _CROWS
_VPR = _D // 16


def _sc_body(x_hbm, e_hbm, o_hbm, xb0, eb0, xb1, eb1, sx0, se0, so0, sx1, se1, so1):
    wid = lax.axis_index("s") * _NC + lax.axis_index("c")
    base = wid * _W_ROWS
    xbs = (xb0, xb1)
    ebs = (eb0, eb1)
    sxs = (sx0, sx1)
    ses = (se0, se1)
    sos = (so0, so1)

    def start_in(g, b):
        r0 = base + g * _CROWS
        pltpu.make_async_copy(
            x_hbm.at[0, pl.ds(_TC_ROWS + r0, _CROWS)], xbs[b], sxs[b]).start()
        pltpu.make_async_copy(
            e_hbm.at[pl.ds(_TC_ROWS + r0, _CROWS)], ebs[b], ses[b]).start()

    def wait_in(b):
        pltpu.make_async_copy(x_hbm.at[0, pl.ds(0, _CROWS)], xbs[b], sxs[b]).wait()
        pltpu.make_async_copy(e_hbm.at[pl.ds(0, _CROWS)], ebs[b], ses[b]).wait()

    def start_out(g, b):
        r0 = base + g * _CROWS
        pltpu.make_async_copy(xbs[b], o_hbm.at[0, pl.ds(r0, _CROWS)], sos[b]).start()

    def wait_out(b):
        pltpu.make_async_copy(xbs[b], o_hbm.at[0, pl.ds(0, _CROWS)], sos[b]).wait()

    def compute(b):
        xb, eb = xbs[b], ebs[b]

        def vec_body(i, c):
            s = pl.ds(i * 16, 16)
            for r in range(_CROWS):
                xb[r, s] = xb[r, s] + eb[r, s]
            return c

        lax.fori_loop(0, _VPR, vec_body, 0)

    start_in(0, 0)
    for g in range(_N_CHUNKS):
        b = g % 2
        if g + 1 < _N_CHUNKS:
            if g >= 1:
                wait_out(1 - b)
            start_in(g + 1, 1 - b)
        wait_in(b)
        compute(b)
        start_out(g, b)
    wait_out(0)
    wait_out(1)


_sc_add = functools.partial(
    pl.kernel,
    mesh=plsc.VectorSubcoreMesh(core_axis_name="c", subcore_axis_name="s"),
    out_type=jax.ShapeDtypeStruct((1, _SC_ROWS, _D), jnp.float32),
    scratch_types=[
        pltpu.VMEM((_CROWS, _D), jnp.float32),
        pltpu.VMEM((_CROWS, _D), jnp.float32),
        pltpu.VMEM((_CROWS, _D), jnp.float32),
        pltpu.VMEM((_CROWS, _D), jnp.float32),
        pltpu.SemaphoreType.DMA,
        pltpu.SemaphoreType.DMA,
        pltpu.SemaphoreType.DMA,
        pltpu.SemaphoreType.DMA,
        pltpu.SemaphoreType.DMA,
        pltpu.SemaphoreType.DMA,
    ],
)(_sc_body)


_TC_BLK = 1024


def _tc_body(x_ref, e_ref, o_ref):
    o_ref[...] = x_ref[...] + e_ref[...]


_tc_add = pl.pallas_call(
    _tc_body,
    grid=(_TC_ROWS // _TC_BLK,),
    in_specs=[
        pl.BlockSpec((1, _TC_BLK, _D), lambda i: (0, i, 0)),
        pl.BlockSpec((_TC_BLK, _D), lambda i: (i, 0)),
    ],
    out_specs=pl.BlockSpec((1, _TC_BLK, _D), lambda i: (0, i, 0)),
    out_shape=jax.ShapeDtypeStruct((1, _S, _D), jnp.float32),
)


def kernel(x, encoding):
    sc_out = _sc_add(x, encoding)
    tc_out = _tc_add(x, encoding)
    return lax.dynamic_update_slice(tc_out, sc_out, (0, _TC_ROWS, 0))


# speedup vs baseline: 4.5590x; 1.0737x over previous
import functools

import jax
import jax.numpy as jnp
from jax import lax
from jax.experimental import pallas as pl
from jax.experimental.pallas import tpu as pltpu
from jax.experimental.pallas import tpu_sc as plsc

_S = 8192
_D = 1024
_NC = 2
_NS = 16
_NW = _NC * _NS

_TC_ROWS = 7168
_SC_ROWS = _S - _TC_ROWS
_W_ROWS = _SC_ROWS // _NW
_CROWS = 16
_N_CHUNKS = _W_ROWS // _CROWS
_VPR = _D // 16


def _sc_body(x_hbm, e_hbm, o_hbm, xb0, eb0, xb1, eb1, sx0, se0, so0, sx1, se1, so1):
    wid = lax.axis_index("s") * _NC + lax.axis_index("c")
    base = wid * _W_ROWS
    xbs = (xb0, xb1)
    ebs = (eb0, eb1)
    sxs = (sx0, sx1)
    ses = (se0, se1)
    sos = (so0, so1)

    def start_in(g, b):
        r0 = base + g * _CROWS
        pltpu.make_async_copy(
            x_hbm.at[0, pl.ds(_TC_ROWS + r0, _CROWS)], xbs[b], sxs[b]).start()
        pltpu.make_async_copy(
            e_hbm.at[pl.ds(_TC_ROWS + r0, _CROWS)], ebs[b], ses[b]).start()

    def wait_in(b):
        pltpu.make_async_copy(x_hbm.at[0, pl.ds(0, _CROWS)], xbs[b], sxs[b]).wait()
        pltpu.make_async_copy(e_hbm.at[pl.ds(0, _CROWS)], ebs[b], ses[b]).wait()

    def start_out(g, b):
        r0 = base + g * _CROWS
        pltpu.make_async_copy(xbs[b], o_hbm.at[0, pl.ds(r0, _CROWS)], sos[b]).start()

    def wait_out(b):
        pltpu.make_async_copy(xbs[b], o_hbm.at[0, pl.ds(0, _CROWS)], sos[b]).wait()

    def compute(b):
        xb, eb = xbs[b], ebs[b]

        def vec_body(i, c):
            s = pl.ds(i * 16, 16)
            for r in range(_CROWS):
                xb[r, s] = xb[r, s] + eb[r, s]
            return c

        lax.fori_loop(0, _VPR, vec_body, 0)

    start_in(0, 0)
    for g in range(_N_CHUNKS):
        b = g % 2
        if g + 1 < _N_CHUNKS:
            if g >= 1:
                wait_out(1 - b)
            start_in(g + 1, 1 - b)
        wait_in(b)
        compute(b)
        start_out(g, b)
    wait_out(0)
    wait_out(1)


_sc_add = functools.partial(
    pl.kernel,
    mesh=plsc.VectorSubcoreMesh(core_axis_name="c", subcore_axis_name="s"),
    out_type=jax.ShapeDtypeStruct((1, _SC_ROWS, _D), jnp.float32),
    scratch_types=[
        pltpu.VMEM((_CROWS, _D), jnp.float32),
        pltpu.VMEM((_CROWS, _D), jnp.float32),
        pltpu.VMEM((_CROWS, _D), jnp.float32),
        pltpu.VMEM((_CROWS, _D), jnp.float32),
        pltpu.SemaphoreType.DMA,
        pltpu.SemaphoreType.DMA,
        pltpu.SemaphoreType.DMA,
        pltpu.SemaphoreType.DMA,
        pltpu.SemaphoreType.DMA,
        pltpu.SemaphoreType.DMA,
    ],
)(_sc_body)


_TC_BLK = 1024


def _tc_body(x_ref, e_ref, o_ref):
    o_ref[...] = x_ref[...] + e_ref[...]


_tc_add = pl.pallas_call(
    _tc_body,
    grid=(_TC_ROWS // _TC_BLK,),
    in_specs=[
        pl.BlockSpec((1, _TC_BLK, _D), lambda i: (0, i, 0)),
        pl.BlockSpec((_TC_BLK, _D), lambda i: (i, 0)),
    ],
    out_specs=pl.BlockSpec((1, _TC_BLK, _D), lambda i: (0, i, 0)),
    out_shape=jax.ShapeDtypeStruct((1, _S, _D), jnp.float32),
)


def kernel(x, encoding):
    sc_out = _sc_add(x, encoding)
    tc_out = _tc_add(x, encoding)
    return lax.dynamic_update_slice(tc_out, sc_out, (0, _TC_ROWS, 0))
